# baseline (speedup 1.0000x reference)
import functools

import jax
import jax.numpy as jnp
from jax import lax
from jax.experimental import pallas as pl
from jax.experimental.pallas import tpu as pltpu
from jax.experimental.pallas import tpu_sc as plsc

WALK_LEN = 50
BATCH = 4096
D = 64
C = 128
NC, NS = 2, 16
NW = NC * NS


def _sc_embed(nidx, eidx, ntab, etab, npos_f, epos_f):
    mesh = plsc.VectorSubcoreMesh(core_axis_name="c", subcore_axis_name="s")

    @functools.partial(
        pl.kernel,
        mesh=mesh,
        compiler_params=pltpu.CompilerParams(use_tc_tiling_on_sc=False),
        out_type=jax.ShapeDtypeStruct((2 * WALK_LEN, BATCH, D), jnp.float32),
        scratch_types=[
            pltpu.VMEM((WALK_LEN, C), jnp.int32),
            pltpu.VMEM((C, D), jnp.float32),
            pltpu.VMEM((C, D), jnp.float32),
            pltpu.VMEM((WALK_LEN * D,), jnp.float32),
            pltpu.VMEM((WALK_LEN * D,), jnp.float32),
            pltpu.SemaphoreType.DMA,
            pltpu.SemaphoreType.DMA,
        ],
    )
    def k(nidx_hbm, eidx_hbm, ntab_hbm, etab_hbm, npos_hbm, epos_hbm,
          out_hbm, idx_all_v, rows0_v, rows1_v, npos_v, epos_v,
          sem0, sem1):
        wid = lax.axis_index("s") * NC + lax.axis_index("c")
        pltpu.sync_copy(npos_hbm, npos_v)
        pltpu.sync_copy(epos_hbm, epos_v)
        bufs = ((rows0_v, sem0), (rows1_v, sem1))

        def do_table(idx_hbm, tab_hbm, pos_v, out_l_off):
            pltpu.sync_copy(idx_hbm.at[:, pl.ds(wid * C, C)], idx_all_v)

            def start(l, b):
                rows_v, sem = bufs[b]
                return pltpu.async_copy(tab_hbm.at[idx_all_v.at[l]],
                                        rows_v, sem)

            def process(l, b):
                rows_v, sem = bufs[b]
                pltpu.make_async_copy(tab_hbm.at[idx_all_v.at[l]], rows_v,
                                      sem).wait()
                pos_vecs = [pos_v[pl.ds(D * l + 16 * d4, 16)]
                            for d4 in range(D // 16)]

                @plsc.parallel_loop(0, C, unroll=8)
                def _(r):
                    for d4 in range(D // 16):
                        sl = pl.ds(16 * d4, 16)
                        rows_v[r, sl] = rows_v[r, sl] + pos_vecs[d4]
                pltpu.sync_copy(
                    rows_v,
                    out_hbm.at[out_l_off + l, pl.ds(wid * C, C), :])

            start(0, 0)

            def body(kk, _):
                start(2 * kk + 1, 1)
                process(2 * kk, 0)
                start(2 * kk + 2, 0)
                process(2 * kk + 1, 1)
                return 0

            lax.fori_loop(0, WALK_LEN // 2 - 1, body, 0)
            start(WALK_LEN - 1, 1)
            process(WALK_LEN - 2, 0)
            process(WALK_LEN - 1, 1)

        do_table(nidx_hbm, ntab_hbm, npos_v, 0)
        do_table(eidx_hbm, etab_hbm, epos_v, WALK_LEN)

    return k(nidx, eidx, ntab, etab, npos_f, epos_f)


def kernel(node_idx, edge_idx, node_table, edge_table, node_pos, edge_pos):
    nidx = node_idx.astype(jnp.int32)
    eidx = edge_idx.astype(jnp.int32)
    return _sc_embed(nidx, eidx, node_table, edge_table,
                     node_pos.reshape(-1), edge_pos.reshape(-1))


# speedup vs baseline: 1.0024x; 1.0024x over previous
import functools

import jax
import jax.numpy as jnp
from jax import lax
from jax.experimental import pallas as pl
from jax.experimental.pallas import tpu as pltpu
from jax.experimental.pallas import tpu_sc as plsc

WALK_LEN = 50
BATCH = 4096
D = 64
C = 128
NC, NS = 2, 16
NW = NC * NS


def _sc_embed(nidx, eidx, ntab, etab, npos_f, epos_f):
    mesh = plsc.VectorSubcoreMesh(core_axis_name="c", subcore_axis_name="s")

    @functools.partial(
        pl.kernel,
        mesh=mesh,
        compiler_params=pltpu.CompilerParams(use_tc_tiling_on_sc=False),
        out_type=jax.ShapeDtypeStruct((2 * WALK_LEN * BATCH, D), jnp.float32),
        scratch_types=[
            pltpu.VMEM((WALK_LEN, C), jnp.int32),
            pltpu.VMEM((C, D), jnp.float32),
            pltpu.VMEM((C, D), jnp.float32),
            pltpu.VMEM((WALK_LEN * D,), jnp.float32),
            pltpu.VMEM((WALK_LEN * D,), jnp.float32),
            pltpu.SemaphoreType.DMA,
            pltpu.SemaphoreType.DMA,
        ],
    )
    def k(nidx_hbm, eidx_hbm, ntab_hbm, etab_hbm, npos_hbm, epos_hbm,
          out_hbm, idx_all_v, rows0_v, rows1_v, npos_v, epos_v,
          sem0, sem1):
        wid = lax.axis_index("s") * NC + lax.axis_index("c")
        pltpu.sync_copy(npos_hbm, npos_v)
        pltpu.sync_copy(epos_hbm, epos_v)
        bufs = ((rows0_v, sem0), (rows1_v, sem1))

        def do_table(idx_hbm, tab_hbm, pos_v, out_row_off):
            pltpu.sync_copy(idx_hbm.at[:, pl.ds(wid * C, C)], idx_all_v)

            def start(l, b):
                rows_v, sem = bufs[b]
                return pltpu.async_copy(tab_hbm.at[idx_all_v.at[l]],
                                        rows_v, sem)

            def process(l, b):
                rows_v, sem = bufs[b]
                pltpu.make_async_copy(tab_hbm.at[idx_all_v.at[l]], rows_v,
                                      sem).wait()
                pos_vecs = [pos_v[pl.ds(D * l + 16 * d4, 16)]
                            for d4 in range(D // 16)]

                @plsc.parallel_loop(0, C, unroll=8)
                def _(r):
                    for d4 in range(D // 16):
                        sl = pl.ds(16 * d4, 16)
                        rows_v[r, sl] = rows_v[r, sl] + pos_vecs[d4]
                pltpu.sync_copy(
                    rows_v,
                    out_hbm.at[pl.ds(out_row_off + l * BATCH + wid * C, C)])

            start(0, 0)

            def body(kk, _):
                start(2 * kk + 1, 1)
                process(2 * kk, 0)
                start(2 * kk + 2, 0)
                process(2 * kk + 1, 1)
                return 0

            lax.fori_loop(0, WALK_LEN // 2 - 1, body, 0)
            start(WALK_LEN - 1, 1)
            process(WALK_LEN - 2, 0)
            process(WALK_LEN - 1, 1)

        do_table(nidx_hbm, ntab_hbm, npos_v, 0)
        do_table(eidx_hbm, etab_hbm, epos_v, WALK_LEN * BATCH)

    return k(nidx, eidx, ntab, etab, npos_f, epos_f)


def kernel(node_idx, edge_idx, node_table, edge_table, node_pos, edge_pos):
    nidx = node_idx.astype(jnp.int32)
    eidx = edge_idx.astype(jnp.int32)
    out = _sc_embed(nidx, eidx, node_table, edge_table,
                    node_pos.reshape(-1), edge_pos.reshape(-1))
    return out.reshape(2 * WALK_LEN, BATCH, D)
